# baseline (speedup 1.0000x reference)
import functools

import jax
import jax.numpy as jnp
from jax import lax
from jax.experimental import pallas as pl
from jax.experimental.pallas import tpu as pltpu
from jax.experimental.pallas import tpu_sc as plsc

_LANES = 16
_BLK = 128
_LNC = (-1.8895877110388932, 3.385218192432659, -2.331779405969581,
        1.1162195608328185, -0.33073590701041583, 0.05458546334426203,
        -0.003832756714011353)


def _proj_body(w_ref, z_ref, b_ref, out_ref):
    w = w_ref[...]
    z = z_ref[...]
    h = w.shape[1] // 2
    dn = (((1,), (1,)), ((), ()))
    s = lax.dot_general(w[:, :h], z, dn, preferred_element_type=jnp.float32)
    d = lax.dot_general(w[:, h:], z, dn, preferred_element_type=jnp.float32)
    sb = s + b_ref[...][:, None]
    pad = jnp.zeros((2, s.shape[1]), jnp.float32)
    out_ref[...] = jnp.concatenate([sb, d, pad], axis=0)


def _make_sc_gather(n_nodes, n_edges):
    info = plsc.get_sparse_core_info()
    nc, ns = info.num_cores, info.num_subcores
    nw = nc * ns
    n_blocks = n_edges // _BLK
    base_blocks = n_blocks // nw
    n_extra = n_blocks - base_blocks * nw
    e_per_w = base_blocks * _BLK
    mesh = plsc.VectorSubcoreMesh(core_axis_name="c", subcore_axis_name="s")

    @functools.partial(
        pl.kernel,
        out_type=jax.ShapeDtypeStruct((3, n_edges), jnp.float32),
        mesh=mesh,
        scratch_types=[
            pltpu.VMEM((6 * n_nodes,), jnp.float32),
            pltpu.VMEM((base_blocks, 2, _BLK), jnp.int32),
            pltpu.VMEM((3, e_per_w), jnp.float32),
            pltpu.VMEM((1, 2, _BLK), jnp.int32),
            pltpu.VMEM((3, _BLK), jnp.float32),
            pltpu.SemaphoreType.DMA,
            pltpu.SemaphoreType.DMA,
        ],
        compiler_params=pltpu.CompilerParams(needs_layout_passes=False),
    )
    def sc_gather(tab_hbm, ei_hbm, out_hbm,
                  tab_v, ei_v, out_v, eit_v, outt_v, sem0, sem1):
        wid = lax.axis_index("s") * nc + lax.axis_index("c")
        bbase = wid * base_blocks
        cp_tab = pltpu.async_copy(tab_hbm.at[pl.ds(0, 6 * n_nodes)], tab_v,
                                  sem0)
        cp_idx = pltpu.async_copy(ei_hbm.at[pl.ds(bbase, base_blocks)], ei_v,
                                  sem1)
        cp_tab.wait()
        cp_idx.wait()

        def compute16(ev0, ev1, o_ref, col):
            s0 = plsc.load_gather(tab_v, [ev0])
            s1 = plsc.load_gather(tab_v, [ev0 + n_nodes])
            s2 = plsc.load_gather(tab_v, [ev0 + 2 * n_nodes])
            d0 = plsc.load_gather(tab_v, [ev1 + 3 * n_nodes])
            d1 = plsc.load_gather(tab_v, [ev1 + 4 * n_nodes])
            d2 = plsc.load_gather(tab_v, [ev1 + 5 * n_nodes])
            l0, l1, l2 = s0 + d0, s1 + d1, s2 + d2
            m = jnp.maximum(jnp.maximum(l0, l1), l2)
            ssum = (jnp.exp(l0 - m) + jnp.exp(l1 - m) + jnp.exp(l2 - m))
            p = _LNC[6]
            for c in (_LNC[5], _LNC[4], _LNC[3], _LNC[2], _LNC[1], _LNC[0]):
                p = p * ssum + c
            lse = m + p
            sl = pl.ds(col, _LANES)
            o_ref[0, sl] = l0 - lse
            o_ref[1, sl] = l1 - lse
            o_ref[2, sl] = l2 - lse

        gpb = _BLK // _LANES

        @plsc.parallel_loop(0, base_blocks * gpb, unroll=16)
        def _main(i):
            b = i // gpb
            g = i % gpb
            ev0 = ei_v[b, 0, pl.ds(g * _LANES, _LANES)]
            ev1 = ei_v[b, 1, pl.ds(g * _LANES, _LANES)]
            compute16(ev0, ev1, out_v, i * _LANES)
        cp_out = pltpu.async_copy(
            out_v, out_hbm.at[:, pl.ds(bbase * _BLK, e_per_w)], sem1)

        @pl.when(wid >= nw - n_extra)
        def _tail():
            tb = base_blocks * nw + (wid - (nw - n_extra))
            pltpu.async_copy(ei_hbm.at[pl.ds(tb, 1)], eit_v, sem0).wait()

            @plsc.parallel_loop(0, gpb, unroll=8)
            def _tail_loop(g):
                ev0 = eit_v[0, 0, pl.ds(g * _LANES, _LANES)]
                ev1 = eit_v[0, 1, pl.ds(g * _LANES, _LANES)]
                compute16(ev0, ev1, outt_v, g * _LANES)
            pltpu.async_copy(
                outt_v, out_hbm.at[:, pl.ds(tb * _BLK, _BLK)], sem0).wait()

        cp_out.wait()

    return sc_gather


def kernel(z, edge_index, W, b):
    n_nodes, hidden = z.shape
    n_edges = edge_index.shape[1]
    table = pl.pallas_call(
        _proj_body,
        out_shape=jax.ShapeDtypeStruct((8, n_nodes), jnp.float32),
    )(W, z, b)

    ei3 = (edge_index.astype(jnp.int32)
           .reshape(2, n_edges // _BLK, _BLK).transpose(1, 0, 2))
    planes = _make_sc_gather(n_nodes, n_edges)(table.reshape(-1), ei3)
    return planes.T


# speedup vs baseline: 1.3266x; 1.3266x over previous
import functools

import jax
import jax.numpy as jnp
from jax import lax
from jax.experimental import pallas as pl
from jax.experimental.pallas import tpu as pltpu
from jax.experimental.pallas import tpu_sc as plsc

_LANES = 16
_BLK = 128
_LNC = (-1.8895877110388932, 3.385218192432659, -2.331779405969581,
        1.1162195608328185, -0.33073590701041583, 0.05458546334426203,
        -0.003832756714011353)


def _proj_body(w_ref, z_ref, b_ref, out_ref):
    w = w_ref[...]
    z = z_ref[...]
    h = w.shape[1] // 2
    dn = (((1,), (1,)), ((), ()))
    s = lax.dot_general(w[:, :h], z, dn, preferred_element_type=jnp.float32)
    d = lax.dot_general(w[:, h:], z, dn, preferred_element_type=jnp.float32)
    sb = s + b_ref[...][:, None]
    pad = jnp.zeros((2, s.shape[1]), jnp.float32)
    out_ref[...] = jnp.concatenate([sb, d, pad], axis=0)


def _make_sc_gather(n_nodes, n_edges):
    info = plsc.get_sparse_core_info()
    nc, ns = info.num_cores, info.num_subcores
    nw = nc * ns
    n_blocks = n_edges // _BLK
    base_blocks = n_blocks // nw
    n_extra = n_blocks - base_blocks * nw
    e_per_w = base_blocks * _BLK
    mesh = plsc.VectorSubcoreMesh(core_axis_name="c", subcore_axis_name="s")

    @functools.partial(
        pl.kernel,
        out_type=jax.ShapeDtypeStruct((3, n_edges), jnp.float32),
        mesh=mesh,
        scratch_types=[
            pltpu.VMEM((6 * n_nodes,), jnp.float32),
            pltpu.VMEM((base_blocks, 2, _BLK), jnp.int32),
            pltpu.VMEM((3, e_per_w), jnp.float32),
            pltpu.VMEM((1, 2, _BLK), jnp.int32),
            pltpu.VMEM((3, _BLK), jnp.float32),
            pltpu.SemaphoreType.DMA,
            pltpu.SemaphoreType.DMA,
        ],
        compiler_params=pltpu.CompilerParams(needs_layout_passes=False),
    )
    def sc_gather(tab_hbm, ei_hbm, out_hbm,
                  tab_v, ei_v, out_v, eit_v, outt_v, sem0, sem1):
        wid = lax.axis_index("s") * nc + lax.axis_index("c")
        bbase = wid * base_blocks
        cp_tab = pltpu.async_copy(tab_hbm.at[pl.ds(0, 6 * n_nodes)], tab_v,
                                  sem0)
        cp_idx = pltpu.async_copy(ei_hbm.at[pl.ds(bbase, base_blocks)], ei_v,
                                  sem1)
        cp_tab.wait()
        cp_idx.wait()

        def compute16(ev0, ev1, o_ref, col):
            s0 = plsc.load_gather(tab_v, [ev0])
            s1 = plsc.load_gather(tab_v, [ev0 + n_nodes])
            s2 = plsc.load_gather(tab_v, [ev0 + 2 * n_nodes])
            d0 = plsc.load_gather(tab_v, [ev1 + 3 * n_nodes])
            d1 = plsc.load_gather(tab_v, [ev1 + 4 * n_nodes])
            d2 = plsc.load_gather(tab_v, [ev1 + 5 * n_nodes])
            l0, l1, l2 = s0 + d0, s1 + d1, s2 + d2
            m = jnp.maximum(jnp.maximum(l0, l1), l2)
            ssum = (jnp.exp(l0 - m) + jnp.exp(l1 - m) + jnp.exp(l2 - m))
            p = _LNC[6]
            for c in (_LNC[5], _LNC[4], _LNC[3], _LNC[2], _LNC[1], _LNC[0]):
                p = p * ssum + c
            lse = m + p
            sl = pl.ds(col, _LANES)
            o_ref[0, sl] = l0 - lse
            o_ref[1, sl] = l1 - lse
            o_ref[2, sl] = l2 - lse

        gpb = _BLK // _LANES

        @plsc.parallel_loop(0, base_blocks * gpb, unroll=4)
        def _main(i):
            b = i // gpb
            g = i % gpb
            ev0 = ei_v[b, 0, pl.ds(g * _LANES, _LANES)]
            ev1 = ei_v[b, 1, pl.ds(g * _LANES, _LANES)]
            compute16(ev0, ev1, out_v, i * _LANES)
        cp_out = pltpu.async_copy(
            out_v, out_hbm.at[:, pl.ds(bbase * _BLK, e_per_w)], sem1)

        @pl.when(wid >= nw - n_extra)
        def _tail():
            tb = base_blocks * nw + (wid - (nw - n_extra))
            pltpu.async_copy(ei_hbm.at[pl.ds(tb, 1)], eit_v, sem0).wait()

            @plsc.parallel_loop(0, gpb, unroll=8)
            def _tail_loop(g):
                ev0 = eit_v[0, 0, pl.ds(g * _LANES, _LANES)]
                ev1 = eit_v[0, 1, pl.ds(g * _LANES, _LANES)]
                compute16(ev0, ev1, outt_v, g * _LANES)
            pltpu.async_copy(
                outt_v, out_hbm.at[:, pl.ds(tb * _BLK, _BLK)], sem0).wait()

        cp_out.wait()

    return sc_gather


def kernel(z, edge_index, W, b):
    n_nodes, hidden = z.shape
    n_edges = edge_index.shape[1]
    table = pl.pallas_call(
        _proj_body,
        out_shape=jax.ShapeDtypeStruct((8, n_nodes), jnp.float32),
    )(W, z, b)

    ei3 = (edge_index.astype(jnp.int32)
           .reshape(2, n_edges // _BLK, _BLK).transpose(1, 0, 2))
    planes = _make_sc_gather(n_nodes, n_edges)(table.reshape(-1), ei3)
    return planes.T


# speedup vs baseline: 1.3407x; 1.0106x over previous
import functools

import jax
import jax.numpy as jnp
from jax import lax
from jax.experimental import pallas as pl
from jax.experimental.pallas import tpu as pltpu
from jax.experimental.pallas import tpu_sc as plsc

_LANES = 16
_BLK = 128
_LNC = (-1.8895877110388932, 3.385218192432659, -2.331779405969581,
        1.1162195608328185, -0.33073590701041583, 0.05458546334426203,
        -0.003832756714011353)


def _proj_body(w_ref, z_ref, b_ref, out_ref):
    w = w_ref[...]
    z = z_ref[...]
    h = w.shape[1] // 2
    dn = (((1,), (1,)), ((), ()))
    s = lax.dot_general(w[:, :h], z, dn, preferred_element_type=jnp.float32)
    d = lax.dot_general(w[:, h:], z, dn, preferred_element_type=jnp.float32)
    sb = s + b_ref[...][:, None]
    pad = jnp.zeros((2, s.shape[1]), jnp.float32)
    out_ref[...] = jnp.concatenate([sb, d, pad], axis=0)


def _make_sc_gather(n_nodes, n_edges):
    info = plsc.get_sparse_core_info()
    nc, ns = info.num_cores, info.num_subcores
    nw = nc * ns
    n_blocks = n_edges // _BLK
    base_blocks = n_blocks // nw
    n_extra = n_blocks - base_blocks * nw
    e_per_w = base_blocks * _BLK
    mesh = plsc.VectorSubcoreMesh(core_axis_name="c", subcore_axis_name="s")

    @functools.partial(
        pl.kernel,
        out_type=jax.ShapeDtypeStruct((3, n_edges), jnp.float32),
        mesh=mesh,
        scratch_types=[
            pltpu.VMEM((6 * n_nodes,), jnp.float32),
            pltpu.VMEM((base_blocks, 2, _BLK), jnp.int32),
            pltpu.VMEM((3, e_per_w), jnp.float32),
            pltpu.VMEM((1, 2, _BLK), jnp.int32),
            pltpu.VMEM((3, _BLK), jnp.float32),
            pltpu.SemaphoreType.DMA,
            pltpu.SemaphoreType.DMA,
        ],
        compiler_params=pltpu.CompilerParams(needs_layout_passes=False),
    )
    def sc_gather(tab_hbm, ei_hbm, out_hbm,
                  tab_v, ei_v, out_v, eit_v, outt_v, sem0, sem1):
        wid = lax.axis_index("s") * nc + lax.axis_index("c")
        bbase = wid * base_blocks
        cp_tab = pltpu.async_copy(tab_hbm.at[pl.ds(0, 6 * n_nodes)], tab_v,
                                  sem0)
        cp_idx = pltpu.async_copy(ei_hbm.at[pl.ds(bbase, base_blocks)], ei_v,
                                  sem1)
        cp_tab.wait()
        cp_idx.wait()

        def compute16(ev0, ev1, o_ref, col):
            s0 = plsc.load_gather(tab_v, [ev0])
            s1 = plsc.load_gather(tab_v, [ev0 + n_nodes])
            s2 = plsc.load_gather(tab_v, [ev0 + 2 * n_nodes])
            d0 = plsc.load_gather(tab_v, [ev1 + 3 * n_nodes])
            d1 = plsc.load_gather(tab_v, [ev1 + 4 * n_nodes])
            d2 = plsc.load_gather(tab_v, [ev1 + 5 * n_nodes])
            l0, l1, l2 = s0 + d0, s1 + d1, s2 + d2
            m = jnp.maximum(jnp.maximum(l0, l1), l2)
            ssum = (jnp.exp(l0 - m) + jnp.exp(l1 - m) + jnp.exp(l2 - m))
            p = _LNC[6]
            for c in (_LNC[5], _LNC[4], _LNC[3], _LNC[2], _LNC[1], _LNC[0]):
                p = p * ssum + c
            lse = m + p
            sl = pl.ds(col, _LANES)
            o_ref[0, sl] = l0 - lse
            o_ref[1, sl] = l1 - lse
            o_ref[2, sl] = l2 - lse

        gpb = _BLK // _LANES

        @plsc.parallel_loop(0, base_blocks * gpb, unroll=6)
        def _main(i):
            b = i // gpb
            g = i % gpb
            ev0 = ei_v[b, 0, pl.ds(g * _LANES, _LANES)]
            ev1 = ei_v[b, 1, pl.ds(g * _LANES, _LANES)]
            compute16(ev0, ev1, out_v, i * _LANES)
        cp_out = pltpu.async_copy(
            out_v, out_hbm.at[:, pl.ds(bbase * _BLK, e_per_w)], sem1)

        @pl.when(wid >= nw - n_extra)
        def _tail():
            tb = base_blocks * nw + (wid - (nw - n_extra))
            pltpu.async_copy(ei_hbm.at[pl.ds(tb, 1)], eit_v, sem0).wait()

            @plsc.parallel_loop(0, gpb, unroll=8)
            def _tail_loop(g):
                ev0 = eit_v[0, 0, pl.ds(g * _LANES, _LANES)]
                ev1 = eit_v[0, 1, pl.ds(g * _LANES, _LANES)]
                compute16(ev0, ev1, outt_v, g * _LANES)
            pltpu.async_copy(
                outt_v, out_hbm.at[:, pl.ds(tb * _BLK, _BLK)], sem0).wait()

        cp_out.wait()

    return sc_gather


def kernel(z, edge_index, W, b):
    n_nodes, hidden = z.shape
    n_edges = edge_index.shape[1]
    table = pl.pallas_call(
        _proj_body,
        out_shape=jax.ShapeDtypeStruct((8, n_nodes), jnp.float32),
    )(W, z, b)

    ei3 = (edge_index.astype(jnp.int32)
           .reshape(2, n_edges // _BLK, _BLK).transpose(1, 0, 2))
    planes = _make_sc_gather(n_nodes, n_edges)(table.reshape(-1), ei3)
    return planes.T


# speedup vs baseline: 1.3441x; 1.0026x over previous
import functools

import jax
import jax.numpy as jnp
from jax import lax
from jax.experimental import pallas as pl
from jax.experimental.pallas import tpu as pltpu
from jax.experimental.pallas import tpu_sc as plsc

_LANES = 16
_BLK = 128
_LNC = (-1.8895877110388932, 3.385218192432659, -2.331779405969581,
        1.1162195608328185, -0.33073590701041583, 0.05458546334426203,
        -0.003832756714011353)


def _proj_body(w_ref, z_ref, b_ref, out_ref):
    w = w_ref[...]
    z = z_ref[...]
    h = w.shape[1] // 2
    dn = (((1,), (1,)), ((), ()))
    s = lax.dot_general(w[:, :h], z, dn, preferred_element_type=jnp.float32)
    d = lax.dot_general(w[:, h:], z, dn, preferred_element_type=jnp.float32)
    sb = s + b_ref[...][:, None]
    pad = jnp.zeros((2, s.shape[1]), jnp.float32)
    out_ref[...] = jnp.concatenate([sb, d, pad], axis=0)


def _make_sc_gather(n_nodes, n_edges):
    info = plsc.get_sparse_core_info()
    nc, ns = info.num_cores, info.num_subcores
    nw = nc * ns
    n_blocks = n_edges // _BLK
    blocks_per_w = -(-n_blocks // nw)
    last_start = n_blocks - blocks_per_w
    e_per_w = blocks_per_w * _BLK
    mesh = plsc.VectorSubcoreMesh(core_axis_name="c", subcore_axis_name="s")

    @functools.partial(
        pl.kernel,
        out_type=jax.ShapeDtypeStruct((3, n_edges), jnp.float32),
        mesh=mesh,
        scratch_types=[
            pltpu.VMEM((6 * n_nodes,), jnp.float32),
            pltpu.VMEM((blocks_per_w, 2, _BLK), jnp.int32),
            pltpu.VMEM((3, e_per_w), jnp.float32),
            pltpu.SemaphoreType.DMA,
            pltpu.SemaphoreType.DMA,
        ],
        compiler_params=pltpu.CompilerParams(needs_layout_passes=False),
    )
    def sc_gather(tab_hbm, ei_hbm, out_hbm, tab_v, ei_v, out_v, sem0, sem1):
        wid = lax.axis_index("s") * nc + lax.axis_index("c")
        sb = (wid * last_start) // (nw - 1)
        cp_tab = pltpu.async_copy(tab_hbm.at[pl.ds(0, 6 * n_nodes)], tab_v,
                                  sem0)
        cp_idx = pltpu.async_copy(ei_hbm.at[pl.ds(sb, blocks_per_w)], ei_v,
                                  sem1)
        cp_tab.wait()
        cp_idx.wait()

        def compute16(ev0, ev1, o_ref, col):
            s0 = plsc.load_gather(tab_v, [ev0])
            s1 = plsc.load_gather(tab_v, [ev0 + n_nodes])
            s2 = plsc.load_gather(tab_v, [ev0 + 2 * n_nodes])
            d0 = plsc.load_gather(tab_v, [ev1 + 3 * n_nodes])
            d1 = plsc.load_gather(tab_v, [ev1 + 4 * n_nodes])
            d2 = plsc.load_gather(tab_v, [ev1 + 5 * n_nodes])
            l0, l1, l2 = s0 + d0, s1 + d1, s2 + d2
            m = jnp.maximum(jnp.maximum(l0, l1), l2)
            ssum = (jnp.exp(l0 - m) + jnp.exp(l1 - m) + jnp.exp(l2 - m))
            p = _LNC[6]
            for c in (_LNC[5], _LNC[4], _LNC[3], _LNC[2], _LNC[1], _LNC[0]):
                p = p * ssum + c
            lse = m + p
            sl = pl.ds(col, _LANES)
            o_ref[0, sl] = l0 - lse
            o_ref[1, sl] = l1 - lse
            o_ref[2, sl] = l2 - lse

        gpb = _BLK // _LANES

        @plsc.parallel_loop(0, blocks_per_w * gpb, unroll=6)
        def _main(i):
            b = i // gpb
            g = i % gpb
            ev0 = ei_v[b, 0, pl.ds(g * _LANES, _LANES)]
            ev1 = ei_v[b, 1, pl.ds(g * _LANES, _LANES)]
            compute16(ev0, ev1, out_v, i * _LANES)

        pltpu.async_copy(
            out_v, out_hbm.at[:, pl.ds(sb * _BLK, e_per_w)], sem1).wait()

    return sc_gather


def kernel(z, edge_index, W, b):
    n_nodes, hidden = z.shape
    n_edges = edge_index.shape[1]
    table = pl.pallas_call(
        _proj_body,
        out_shape=jax.ShapeDtypeStruct((8, n_nodes), jnp.float32),
    )(W, z, b)

    ei3 = (edge_index.astype(jnp.int32)
           .reshape(2, n_edges // _BLK, _BLK).transpose(1, 0, 2))
    planes = _make_sc_gather(n_nodes, n_edges)(table.reshape(-1), ei3)
    return planes.T
